# baseline (speedup 1.0000x reference)
import functools

import jax
import jax.numpy as jnp
from jax.experimental import pallas as pl
from jax.experimental.pallas import tpu as pltpu

_PREC = jax.lax.Precision.DEFAULT

_SIZES = [128, 128, 256] + [512] * 6 + [256, 128, 128]
_CHUNKS = []
_off = 0
for _sz in _SIZES:
    _CHUNKS.append((_off, _sz))
    _off += _sz
_NA = 4
_NO = 2


def _gnn_body(adj_hbm, w_hbm, f_hbm, out_hbm,
              a_buf, o_buf, w_ref, f_ref, in_sems, out_sems, wf_sem):
    def in_copy(ci):
        off, sz = _CHUNKS[ci]
        return pltpu.make_async_copy(
            adj_hbm.at[pl.ds(off, sz)],
            a_buf.at[ci % _NA, pl.ds(0, sz)],
            in_sems.at[ci % _NA])

    def out_copy(ci):
        off, sz = _CHUNKS[ci]
        return pltpu.make_async_copy(
            o_buf.at[ci % _NO, pl.ds(0, sz)],
            out_hbm.at[pl.ds(off, sz)],
            out_sems.at[ci % _NO])

    w_copy = pltpu.make_async_copy(w_hbm, w_ref, wf_sem)
    f_copy = pltpu.make_async_copy(f_hbm, f_ref, wf_sem)

    n_c = len(_CHUNKS)
    in_copy(0).start()
    w_copy.start()
    f_copy.start()
    for ci in range(1, _NA - 1):
        in_copy(ci).start()
    w_copy.wait()
    f_copy.wait()
    for i in range(n_c):
        if i + _NA - 1 < n_c:
            in_copy(i + _NA - 1).start()
        in_copy(i).wait()
        if i >= _NO:
            out_copy(i - _NO).wait()
        _, sz = _CHUNKS[i]
        a = a_buf[i % _NA, 0:sz]
        p = jnp.dot(a, w_ref[...],
                    preferred_element_type=jnp.float32, precision=_PREC)
        o_buf[i % _NO, 0:sz] = jnp.maximum(
            jnp.dot(p, f_ref[...],
                    preferred_element_type=jnp.float32, precision=_PREC),
            0.0)
        out_copy(i).start()
    out_copy(n_c - 2).wait()
    out_copy(n_c - 1).wait()


@jax.jit
def _gnn(features, adj, weight):
    n, in_f = adj.shape
    out_f = features.shape[0]
    n_out = features.shape[1]
    max_sz = max(_SIZES)
    return pl.pallas_call(
        _gnn_body,
        in_specs=[
            pl.BlockSpec(memory_space=pltpu.MemorySpace.HBM),
            pl.BlockSpec(memory_space=pltpu.MemorySpace.HBM),
            pl.BlockSpec(memory_space=pltpu.MemorySpace.HBM),
        ],
        out_specs=pl.BlockSpec(memory_space=pltpu.MemorySpace.HBM),
        out_shape=jax.ShapeDtypeStruct((n, n_out), jnp.float32),
        scratch_shapes=[
            pltpu.VMEM((_NA, max_sz, in_f), jnp.float32),
            pltpu.VMEM((_NO, max_sz, n_out), jnp.float32),
            pltpu.VMEM((in_f, out_f), jnp.bfloat16),
            pltpu.VMEM((out_f, n_out), jnp.bfloat16),
            pltpu.SemaphoreType.DMA((_NA,)),
            pltpu.SemaphoreType.DMA((_NO,)),
            pltpu.SemaphoreType.DMA,
        ],
    )(adj, weight.astype(jnp.bfloat16), features.astype(jnp.bfloat16))


def kernel(features, adj, weight):
    return _gnn(features, adj, weight)


# speedup vs baseline: 1.1007x; 1.1007x over previous
import jax
import jax.numpy as jnp
from jax.experimental import pallas as pl
from jax.experimental.pallas import tpu as pltpu

_PREC = jax.lax.Precision.DEFAULT

_SIZES = [128, 128, 256] + [512] * 6 + [256, 128, 128]
_CHUNKS = []
_off = 0
for _sz in _SIZES:
    _CHUNKS.append((_off, _sz))
    _off += _sz
_NA = 4
_NO = 2


def _gnn_body(adj_hbm, w_hbm, f_hbm, out_hbm,
              a_buf, o_buf, w_ref, f_ref, in_sems, out_sems, wf_sem):
    def in_copy(ci):
        off, sz = _CHUNKS[ci]
        return pltpu.make_async_copy(
            adj_hbm.at[pl.ds(off, sz)],
            a_buf.at[ci % _NA, pl.ds(0, sz)],
            in_sems.at[ci % _NA])

    def out_copy(ci):
        off, sz = _CHUNKS[ci]
        return pltpu.make_async_copy(
            o_buf.at[ci % _NO, pl.ds(0, sz)],
            out_hbm.at[pl.ds(off, sz)],
            out_sems.at[ci % _NO])

    w_copy = pltpu.make_async_copy(w_hbm, w_ref, wf_sem)
    f_copy = pltpu.make_async_copy(f_hbm, f_ref, wf_sem)

    n_c = len(_CHUNKS)
    in_copy(0).start()
    w_copy.start()
    f_copy.start()
    for ci in range(1, _NA - 1):
        in_copy(ci).start()
    w_copy.wait()
    f_copy.wait()
    for i in range(n_c):
        if i + _NA - 1 < n_c:
            in_copy(i + _NA - 1).start()
        in_copy(i).wait()
        if i >= _NO:
            out_copy(i - _NO).wait()
        _, sz = _CHUNKS[i]
        a = a_buf[i % _NA, 0:sz]
        p = jnp.dot(a, w_ref[...],
                    preferred_element_type=jnp.float32, precision=_PREC)
        o_buf[i % _NO, 0:sz] = jnp.maximum(
            jnp.dot(p, f_ref[...],
                    preferred_element_type=jnp.float32, precision=_PREC),
            0.0)
        out_copy(i).start()
    out_copy(n_c - 2).wait()
    out_copy(n_c - 1).wait()


@jax.jit
def _gnn(features, adj, weight):
    n, in_f = adj.shape
    out_f = features.shape[0]
    n_out = features.shape[1]
    max_sz = max(_SIZES)
    return pl.pallas_call(
        _gnn_body,
        in_specs=[
            pl.BlockSpec(memory_space=pltpu.MemorySpace.HBM),
            pl.BlockSpec(memory_space=pltpu.MemorySpace.HBM),
            pl.BlockSpec(memory_space=pltpu.MemorySpace.HBM),
        ],
        out_specs=pl.BlockSpec(memory_space=pltpu.MemorySpace.HBM),
        out_shape=jax.ShapeDtypeStruct((n, n_out), jnp.float32),
        scratch_shapes=[
            pltpu.VMEM((_NA, max_sz, in_f), jnp.float32),
            pltpu.VMEM((_NO, max_sz, n_out), jnp.float32),
            pltpu.VMEM((in_f, out_f), jnp.float32),
            pltpu.VMEM((out_f, n_out), jnp.float32),
            pltpu.SemaphoreType.DMA((_NA,)),
            pltpu.SemaphoreType.DMA((_NO,)),
            pltpu.SemaphoreType.DMA,
        ],
    )(adj, weight, features)


def kernel(features, adj, weight):
    return _gnn(features, adj, weight)
